# baseline (speedup 1.0000x reference)
import jax
import jax.numpy as jnp
from jax import lax
from jax.experimental import pallas as pl
from jax.experimental.pallas import tpu as pltpu

_INPUT_DIM = 4096
_HIDDEN = 16384
_K = 16
_B = 32
_HB = 512
_NB = _HIDDEN // _HB


def _body(x_ref, encw_ref, encb_ref, decw_ref, decb_ref,
          out_ref, hsp_ref, h_ref, mask_ref, work_ref, run_ref):
    p = pl.program_id(0)
    i = pl.program_id(1)

    @pl.when(p == 0)
    def _enc():
        hb = lax.dot_general(
            x_ref[...], encw_ref[...],
            (((1,), (1,)), ((), ())),
            preferred_element_type=jnp.float32)
        hb = jnp.maximum(hb + encb_ref[:, pl.ds(i * _HB, _HB)], 0.0)
        h_ref[:, pl.ds(i * _HB, _HB)] = hb

        run0 = jnp.where(i == 0, -jnp.inf, run_ref[...])
        work = jnp.concatenate([hb, run0], axis=1)
        tcol = lax.broadcasted_iota(jnp.int32, (_B, _K), 1)

        def mrg(t, carry):
            work, run = carry
            m = jnp.max(work, axis=1, keepdims=True)
            run = jnp.where(tcol == t, m, run)
            work = jnp.where(work == m, -jnp.inf, work)
            return (work, run)

        _, run = lax.fori_loop(0, _K, mrg,
                               (work, jnp.zeros((_B, _K), jnp.float32)))
        run_ref[...] = run

    @pl.when((p == 1) & (i == 0))
    def _topk():
        h = h_ref[...]
        thresh = run_ref[:, _K - 1:_K]
        ge = h >= thresh
        cnt = jnp.sum(ge.astype(jnp.float32), axis=1, keepdims=True)
        allok = jnp.all(cnt == float(_K))

        @pl.when(allok)
        def _fast():
            mask_ref[...] = ge.astype(jnp.float32)

        @pl.when(jnp.logical_not(allok))
        def _exact():
            work_ref[...] = h
            mask_ref[...] = jnp.zeros_like(mask_ref)
            colid = lax.broadcasted_iota(jnp.int32, (_B, _HIDDEN), 1)

            def it(t, carry):
                w = work_ref[...]
                m = jnp.max(w, axis=1, keepdims=True)
                cand = jnp.where(w == m, colid, _HIDDEN)
                amin = jnp.min(cand, axis=1, keepdims=True)
                first = colid == amin
                mask_ref[...] = jnp.where(first, 1.0, mask_ref[...])
                work_ref[...] = jnp.where(first, -jnp.inf, w)
                return carry

            lax.fori_loop(0, _K, it, 0)

    @pl.when(p == 1)
    def _dec():
        hs = h_ref[:, pl.ds(i * _HB, _HB)] * mask_ref[:, pl.ds(i * _HB, _HB)]
        hsp_ref[...] = hs
        contrib = lax.dot_general(
            hs, decw_ref[...],
            (((1,), (1,)), ((), ())),
            preferred_element_type=jnp.float32)

        @pl.when(i == 0)
        def _init():
            out_ref[...] = decb_ref[...] + contrib

        @pl.when(i != 0)
        def _acc():
            out_ref[...] += contrib


def kernel(x, enc_w, enc_b, dec_w, dec_b):
    enc_b2 = enc_b.reshape(1, _HIDDEN)
    dec_b2 = dec_b.reshape(1, _INPUT_DIM)

    out, h_sparse = pl.pallas_call(
        _body,
        grid=(2, _NB),
        in_specs=[
            pl.BlockSpec((_B, _INPUT_DIM), lambda p, i: (0, 0)),
            pl.BlockSpec((_HB, _INPUT_DIM),
                         lambda p, i: (i * (1 - p) + (_NB - 1) * p, 0)),
            pl.BlockSpec((1, _HIDDEN), lambda p, i: (0, 0)),
            pl.BlockSpec((_INPUT_DIM, _HB), lambda p, i: (0, i * p)),
            pl.BlockSpec((1, _INPUT_DIM), lambda p, i: (0, 0)),
        ],
        out_specs=[
            pl.BlockSpec((_B, _INPUT_DIM), lambda p, i: (0, 0)),
            pl.BlockSpec((_B, _HB), lambda p, i: (0, i * p)),
        ],
        out_shape=[
            jax.ShapeDtypeStruct((_B, _INPUT_DIM), jnp.float32),
            jax.ShapeDtypeStruct((_B, _HIDDEN), jnp.float32),
        ],
        scratch_shapes=[
            pltpu.VMEM((_B, _HIDDEN), jnp.float32),
            pltpu.VMEM((_B, _HIDDEN), jnp.float32),
            pltpu.VMEM((_B, _HIDDEN), jnp.float32),
            pltpu.VMEM((_B, _K), jnp.float32),
        ],
        compiler_params=pltpu.CompilerParams(
            dimension_semantics=("arbitrary", "arbitrary"),
        ),
    )(x, enc_w, enc_b2, dec_w, dec_b2)
    return (out, h_sparse)


# speedup vs baseline: 1.0834x; 1.0834x over previous
import jax
import jax.numpy as jnp
from jax import lax
from jax.experimental import pallas as pl
from jax.experimental.pallas import tpu as pltpu

_INPUT_DIM = 4096
_HIDDEN = 16384
_K = 16
_B = 32
_HB = 512
_NB = _HIDDEN // _HB


def _body(x_ref, encw_ref, encb_ref, decw_ref, decb_ref,
          out_ref, hsp_ref, h_ref, mask_ref, work_ref):
    p = pl.program_id(0)
    i = pl.program_id(1)

    @pl.when(p == 0)
    def _enc():
        hb = lax.dot_general(
            x_ref[...], encw_ref[...],
            (((1,), (1,)), ((), ())),
            preferred_element_type=jnp.float32)
        hb = jnp.maximum(hb + encb_ref[:, pl.ds(i * _HB, _HB)], 0.0)
        h_ref[:, pl.ds(i * _HB, _HB)] = hb

    @pl.when((p == 1) & (i == 0))
    def _topk():
        h = h_ref[...]
        work_ref[...] = h

        def vit(t, thresh):
            w = work_ref[...]
            m = jnp.max(w, axis=1, keepdims=True)
            work_ref[...] = jnp.where(w == m, -jnp.inf, w)
            return m

        thresh = lax.fori_loop(0, _K, vit,
                               jnp.zeros((_B, 1), jnp.float32))
        ge = h >= thresh
        cnt = jnp.sum(ge.astype(jnp.float32), axis=1, keepdims=True)
        allok = jnp.all(cnt == float(_K))

        @pl.when(allok)
        def _fast():
            mask_ref[...] = ge.astype(jnp.float32)

        @pl.when(jnp.logical_not(allok))
        def _exact():
            work_ref[...] = h
            mask_ref[...] = jnp.zeros_like(mask_ref)
            colid = lax.broadcasted_iota(jnp.int32, (_B, _HIDDEN), 1)

            def it(t, carry):
                w = work_ref[...]
                m = jnp.max(w, axis=1, keepdims=True)
                cand = jnp.where(w == m, colid, _HIDDEN)
                amin = jnp.min(cand, axis=1, keepdims=True)
                first = colid == amin
                mask_ref[...] = jnp.where(first, 1.0, mask_ref[...])
                work_ref[...] = jnp.where(first, -jnp.inf, w)
                return carry

            lax.fori_loop(0, _K, it, 0)

    @pl.when(p == 1)
    def _dec():
        hs = h_ref[:, pl.ds(i * _HB, _HB)] * mask_ref[:, pl.ds(i * _HB, _HB)]
        hsp_ref[...] = hs
        contrib = lax.dot_general(
            hs, decw_ref[...],
            (((1,), (1,)), ((), ())),
            preferred_element_type=jnp.float32)

        @pl.when(i == 0)
        def _init():
            out_ref[...] = decb_ref[...] + contrib

        @pl.when(i != 0)
        def _acc():
            out_ref[...] += contrib


def kernel(x, enc_w, enc_b, dec_w, dec_b):
    enc_b2 = enc_b.reshape(1, _HIDDEN)
    dec_b2 = dec_b.reshape(1, _INPUT_DIM)

    out, h_sparse = pl.pallas_call(
        _body,
        grid=(2, _NB),
        in_specs=[
            pl.BlockSpec((_B, _INPUT_DIM), lambda p, i: (0, 0)),
            pl.BlockSpec((_HB, _INPUT_DIM),
                         lambda p, i: (i * (1 - p) + (_NB - 1) * p, 0)),
            pl.BlockSpec((1, _HIDDEN), lambda p, i: (0, 0)),
            pl.BlockSpec((_INPUT_DIM, _HB), lambda p, i: (0, i * p)),
            pl.BlockSpec((1, _INPUT_DIM), lambda p, i: (0, 0)),
        ],
        out_specs=[
            pl.BlockSpec((_B, _INPUT_DIM), lambda p, i: (0, 0)),
            pl.BlockSpec((_B, _HB), lambda p, i: (0, i * p)),
        ],
        out_shape=[
            jax.ShapeDtypeStruct((_B, _INPUT_DIM), jnp.float32),
            jax.ShapeDtypeStruct((_B, _HIDDEN), jnp.float32),
        ],
        scratch_shapes=[
            pltpu.VMEM((_B, _HIDDEN), jnp.float32),
            pltpu.VMEM((_B, _HIDDEN), jnp.float32),
            pltpu.VMEM((_B, _HIDDEN), jnp.float32),
        ],
        compiler_params=pltpu.CompilerParams(
            dimension_semantics=("arbitrary", "arbitrary"),
        ),
    )(x, enc_w, enc_b2, dec_w, dec_b2)
    return (out, h_sparse)
